# baseline (speedup 1.0000x reference)
import jax
import jax.numpy as jnp
from jax.experimental import pallas as pl
from jax.experimental.pallas import tpu as pltpu

QN = 100
KTOT = 128 * 128 * 16
CS = 20
THR = 0.2

CHUNK_A = 4096
STEPS_A = KTOT // CHUNK_A

CHUNK_B = 16384
NCH_B = KTOT // CHUNK_B


QE = 104


def _pass_a_body(a_ref, b_ref, s_ref, mask_ref, idx_ref, keep_ref, inter_ref):
    k = pl.program_id(0)

    @pl.when(k == 0)
    def _init():
        inter_ref[...] = jnp.zeros_like(inter_ref)

    pad = jnp.ones((QE - QN, CHUNK_A), jnp.bfloat16)
    ma = jnp.concatenate(
        [(a_ref[...] > 0.5).astype(jnp.bfloat16), pad], axis=0)
    mb = jnp.concatenate(
        [(b_ref[...] > 0.5).astype(jnp.bfloat16), pad], axis=0)
    inter_ref[...] += jax.lax.dot_general(
        ma, mb, (((1,), (1,)), ((), ())), preferred_element_type=jnp.float32)

    sem = s_ref[...]
    mx_rest = jnp.max(sem[1:, :], axis=0, keepdims=True)
    mask_ref[...] = (mx_rest > sem[0:1, :]).astype(jnp.float32)

    @pl.when(k == STEPS_A - 1)
    def _finish():
        inter = inter_ref[:QN, :QN]
        aa = inter_ref[:QN, QN:QN + 1]
        ab = inter_ref[QN:QN + 1, :QN]
        union = aa + ab - inter
        iou = inter / jnp.maximum(union, 1.0)
        mx = jnp.max(iou, axis=1, keepdims=True)
        colid = jax.lax.broadcasted_iota(jnp.int32, (QN, QN), 1)
        cand = jnp.where(iou == mx, colid, QN)
        idx_ref[...] = jnp.min(cand, axis=1, keepdims=True)
        keep_ref[...] = (mx > THR).astype(jnp.int32)


def _pass_a(a2, b2, s2):
    return pl.pallas_call(
        _pass_a_body,
        grid=(STEPS_A,),
        in_specs=[
            pl.BlockSpec((QN, CHUNK_A), lambda k: (0, k)),
            pl.BlockSpec((QN, CHUNK_A), lambda k: (0, k)),
            pl.BlockSpec((CS, CHUNK_A), lambda k: (0, k)),
        ],
        out_specs=[
            pl.BlockSpec((1, CHUNK_A), lambda k: (0, k)),
            pl.BlockSpec((QN, 1), lambda k: (0, 0)),
            pl.BlockSpec((QN, 1), lambda k: (0, 0)),
        ],
        out_shape=[
            jax.ShapeDtypeStruct((1, KTOT), jnp.float32),
            jax.ShapeDtypeStruct((QN, 1), jnp.int32),
            jax.ShapeDtypeStruct((QN, 1), jnp.int32),
        ],
        scratch_shapes=[
            pltpu.VMEM((QE, QE), jnp.float32),
        ],
        compiler_params=pltpu.CompilerParams(
            dimension_semantics=("arbitrary",),
        ),
    )(a2, b2, s2)


ROWS_B = 16
COLS_B = KTOT // ROWS_B
SUB_B = 8
NSUB_B = ROWS_B // SUB_B


def _pass_b_body(idx_ref, keep_ref, a_ref, g_ref, m_ref, out_ref):
    q = pl.program_id(0)
    scale = keep_ref[q].astype(jnp.float32) * 0.5
    out_ref[...] = (a_ref[...] + g_ref[...]) * (m_ref[...] * scale)


def _pass_b(idx1, keep1, a3, b3, mask3):
    grid_spec = pltpu.PrefetchScalarGridSpec(
        num_scalar_prefetch=2,
        grid=(QN, NSUB_B),
        in_specs=[
            pl.BlockSpec((1, SUB_B, COLS_B), lambda q, c, idx, keep: (q, c, 0)),
            pl.BlockSpec((1, SUB_B, COLS_B),
                         lambda q, c, idx, keep: (idx[q], c, 0)),
            pl.BlockSpec((1, SUB_B, COLS_B), lambda q, c, idx, keep: (0, c, 0)),
        ],
        out_specs=pl.BlockSpec((1, SUB_B, COLS_B),
                               lambda q, c, idx, keep: (q, c, 0)),
    )
    return pl.pallas_call(
        _pass_b_body,
        grid_spec=grid_spec,
        out_shape=jax.ShapeDtypeStruct((QN, ROWS_B, COLS_B), jnp.float32),
        compiler_params=pltpu.CompilerParams(
            dimension_semantics=("arbitrary", "arbitrary"),
        ),
    )(idx1, keep1, a3, b3, mask3)


def kernel(anchor_query_prob, aux_query_prob, anchor_voxel_prob,
           aux_voxel_prob, ensemble_sem_prob_dense):
    del anchor_query_prob, aux_query_prob
    a2 = anchor_voxel_prob.reshape(QN, KTOT)
    b2 = aux_voxel_prob.reshape(QN, KTOT)
    s2 = ensemble_sem_prob_dense.reshape(CS, KTOT)

    mask, idx, keep = _pass_a(a2, b2, s2)
    out = _pass_b(idx.reshape(QN), keep.reshape(QN),
                  a2.reshape(QN, ROWS_B, COLS_B),
                  b2.reshape(QN, ROWS_B, COLS_B),
                  mask.reshape(1, ROWS_B, COLS_B))
    return out.reshape(QN, 128, 128, 16)


# speedup vs baseline: 1.1506x; 1.1506x over previous
import jax
import jax.numpy as jnp
from jax.experimental import pallas as pl
from jax.experimental.pallas import tpu as pltpu

QN = 100
SUBS = 128
LANES = 2048
CS = 20
THR = 0.2

QE = 104

SB_A = 8
STEPS_A = SUBS // SB_A

SB_B = 64
NSUB_B = SUBS // SB_B


def _pass_a_body(a_ref, b_ref, s_ref, mask_ref, idx_ref, keep_ref, inter_ref):
    k = pl.program_id(0)

    @pl.when(k == 0)
    def _init():
        inter_ref[...] = jnp.zeros_like(inter_ref)

    pad = jnp.ones((QE - QN, LANES), jnp.bfloat16)
    acc = jnp.zeros((QE, QE), jnp.float32)
    for s in range(SB_A):
        ma = jnp.concatenate(
            [(a_ref[:, s, :] > 0.5).astype(jnp.bfloat16), pad], axis=0)
        mb = jnp.concatenate(
            [(b_ref[:, s, :] > 0.5).astype(jnp.bfloat16), pad], axis=0)
        acc += jax.lax.dot_general(
            ma, mb, (((1,), (1,)), ((), ())),
            preferred_element_type=jnp.float32)
        sem = s_ref[:, s, :]
        mx_rest = jnp.max(sem[1:, :], axis=0, keepdims=True)
        mask_ref[:, s, :] = (mx_rest > sem[0:1, :]).astype(jnp.float32)
    inter_ref[...] += acc

    @pl.when(k == STEPS_A - 1)
    def _finish():
        inter = inter_ref[:QN, :QN]
        aa = inter_ref[:QN, QN:QN + 1]
        ab = inter_ref[QN:QN + 1, :QN]
        union = aa + ab - inter
        iou = inter / jnp.maximum(union, 1.0)
        mx = jnp.max(iou, axis=1, keepdims=True)
        colid = jax.lax.broadcasted_iota(jnp.int32, (QN, QN), 1)
        cand = jnp.where(iou == mx, colid, QN)
        idx_ref[...] = jnp.min(cand, axis=1, keepdims=True)
        keep_ref[...] = (mx > THR).astype(jnp.int32)


def _pass_a(a3, b3, s3):
    return pl.pallas_call(
        _pass_a_body,
        grid=(STEPS_A,),
        in_specs=[
            pl.BlockSpec((QN, SB_A, LANES), lambda k: (0, k, 0)),
            pl.BlockSpec((QN, SB_A, LANES), lambda k: (0, k, 0)),
            pl.BlockSpec((CS, SB_A, LANES), lambda k: (0, k, 0)),
        ],
        out_specs=[
            pl.BlockSpec((1, SB_A, LANES), lambda k: (0, k, 0)),
            pl.BlockSpec((QN, 1), lambda k: (0, 0)),
            pl.BlockSpec((QN, 1), lambda k: (0, 0)),
        ],
        out_shape=[
            jax.ShapeDtypeStruct((1, SUBS, LANES), jnp.float32),
            jax.ShapeDtypeStruct((QN, 1), jnp.int32),
            jax.ShapeDtypeStruct((QN, 1), jnp.int32),
        ],
        scratch_shapes=[
            pltpu.VMEM((QE, QE), jnp.float32),
        ],
        compiler_params=pltpu.CompilerParams(
            dimension_semantics=("arbitrary",),
        ),
    )(a3, b3, s3)


def _pass_b_body(idx_ref, keep_ref, a_ref, g_ref, m_ref, out_ref):
    q = pl.program_id(0)
    scale = keep_ref[q].astype(jnp.float32) * 0.5
    out_ref[...] = (a_ref[...] + g_ref[...]) * (m_ref[...] * scale)


def _pass_b(idx1, keep1, a3, b3, mask3):
    grid_spec = pltpu.PrefetchScalarGridSpec(
        num_scalar_prefetch=2,
        grid=(QN, NSUB_B),
        in_specs=[
            pl.BlockSpec((1, SB_B, LANES), lambda q, c, idx, keep: (q, c, 0)),
            pl.BlockSpec((1, SB_B, LANES),
                         lambda q, c, idx, keep: (idx[q], c, 0)),
            pl.BlockSpec((1, SB_B, LANES), lambda q, c, idx, keep: (0, c, 0)),
        ],
        out_specs=pl.BlockSpec((1, SB_B, LANES),
                               lambda q, c, idx, keep: (q, c, 0)),
    )
    return pl.pallas_call(
        _pass_b_body,
        grid_spec=grid_spec,
        out_shape=jax.ShapeDtypeStruct((QN, SUBS, LANES), jnp.float32),
        compiler_params=pltpu.CompilerParams(
            dimension_semantics=("arbitrary", "arbitrary"),
        ),
    )(idx1, keep1, a3, b3, mask3)


def kernel(anchor_query_prob, aux_query_prob, anchor_voxel_prob,
           aux_voxel_prob, ensemble_sem_prob_dense):
    del anchor_query_prob, aux_query_prob
    a3 = anchor_voxel_prob.reshape(QN, SUBS, LANES)
    b3 = aux_voxel_prob.reshape(QN, SUBS, LANES)
    s3 = ensemble_sem_prob_dense.reshape(CS, SUBS, LANES)

    mask, idx, keep = _pass_a(a3, b3, s3)
    out = _pass_b(idx.reshape(QN), keep.reshape(QN), a3, b3, mask)
    return out.reshape(QN, 128, 128, 16)


# speedup vs baseline: 2.3979x; 2.0840x over previous
import jax
import jax.numpy as jnp
from jax.experimental import pallas as pl
from jax.experimental.pallas import tpu as pltpu

QN = 100
SUBS = 2048
LANES = 128
KTOT = SUBS * LANES
CS = 20
THR = 0.2

NCH_A = 32
KC_A = KTOT // NCH_A
SB_A = SUBS // NCH_A

SB_B = 512
NSUB_B = SUBS // SB_B

QPAD = 128


def _pass_a_body(ta_ref, tb_ref, s_ref, mask_ref, idx_ref, scale_ref,
                 inter_ref, aa_ref, ab_ref):
    k = pl.program_id(0)

    @pl.when(k == 0)
    def _init():
        inter_ref[...] = jnp.zeros_like(inter_ref)
        aa_ref[...] = jnp.zeros_like(aa_ref)
        ab_ref[...] = jnp.zeros_like(ab_ref)

    ta = ta_ref[...]
    tb = tb_ref[...]
    ones = jnp.ones((KC_A, 8), jnp.int8)
    dn = (((0,), (0,)), ((), ()))
    inter_ref[...] += jax.lax.dot_general(
        ta, tb, dn, preferred_element_type=jnp.int32)
    aa_ref[...] += jax.lax.dot_general(
        ta, ones, dn, preferred_element_type=jnp.int32)
    ab_ref[...] += jax.lax.dot_general(
        ones, tb, dn, preferred_element_type=jnp.int32)

    sem = s_ref[...]
    mx_rest = jnp.max(sem[1:], axis=0, keepdims=True)
    mask_ref[...] = (mx_rest > sem[0:1]).astype(jnp.float32)

    @pl.when(k == NCH_A - 1)
    def _finish():
        inter = inter_ref[...].astype(jnp.float32)
        union = (aa_ref[:, 0:1] + ab_ref[0:1, :]).astype(jnp.float32) - inter
        iou = inter / jnp.maximum(union, 1.0)
        mx = jnp.max(iou, axis=1, keepdims=True)
        colid = jax.lax.broadcasted_iota(jnp.int32, (QN, QN), 1)
        cand = jnp.where(iou == mx, colid, QN)
        idx = jnp.min(cand, axis=1, keepdims=True)
        scale = jnp.where(mx > THR, 0.5, 0.0)
        zi = jnp.zeros((QPAD - QN, 1), jnp.int32)
        zf = jnp.zeros((QPAD - QN, 1), jnp.float32)
        idx_ref[...] = jnp.concatenate([idx, zi], axis=0)
        scale_ref[...] = jnp.concatenate([scale, zf], axis=0)


def _pass_a(ta, tb, s3):
    return pl.pallas_call(
        _pass_a_body,
        grid=(NCH_A,),
        in_specs=[
            pl.BlockSpec((KC_A, QN), lambda k: (k, 0)),
            pl.BlockSpec((KC_A, QN), lambda k: (k, 0)),
            pl.BlockSpec((CS, SB_A, LANES), lambda k: (0, k, 0)),
        ],
        out_specs=[
            pl.BlockSpec((1, SB_A, LANES), lambda k: (0, k, 0)),
            pl.BlockSpec((QPAD, 1), lambda k: (0, 0)),
            pl.BlockSpec((QPAD, 1), lambda k: (0, 0)),
        ],
        out_shape=[
            jax.ShapeDtypeStruct((1, SUBS, LANES), jnp.float32),
            jax.ShapeDtypeStruct((QPAD, 1), jnp.int32),
            jax.ShapeDtypeStruct((QPAD, 1), jnp.float32),
        ],
        scratch_shapes=[
            pltpu.VMEM((QN, QN), jnp.int32),
            pltpu.VMEM((QN, 8), jnp.int32),
            pltpu.VMEM((8, QN), jnp.int32),
        ],
        compiler_params=pltpu.CompilerParams(
            dimension_semantics=("arbitrary",),
        ),
    )(ta, tb, s3)


def _pass_b_body(idx_s, scale_s, a_ref, b_ref, m_ref, o_ref):
    q = pl.program_id(0)
    del idx_s
    s = scale_s[q]
    o_ref[...] = (a_ref[...] + b_ref[...]) * (m_ref[...] * s)


def _pass_b(idx, scale, a3, b3, mask):
    grid_spec = pltpu.PrefetchScalarGridSpec(
        num_scalar_prefetch=2,
        grid=(QN, NSUB_B),
        in_specs=[
            pl.BlockSpec((1, SB_B, LANES), lambda q, j, idx, sc: (q, j, 0)),
            pl.BlockSpec(
                (1, SB_B, LANES), lambda q, j, idx, sc: (idx[q], j, 0)),
            pl.BlockSpec((1, SB_B, LANES), lambda q, j, idx, sc: (0, j, 0)),
        ],
        out_specs=pl.BlockSpec(
            (1, SB_B, LANES), lambda q, j, idx, sc: (q, j, 0)),
    )
    return pl.pallas_call(
        _pass_b_body,
        grid_spec=grid_spec,
        out_shape=jax.ShapeDtypeStruct((QN, SUBS, LANES), jnp.float32),
        compiler_params=pltpu.CompilerParams(
            dimension_semantics=("arbitrary", "arbitrary"),
        ),
    )(idx, scale, a3, b3, mask)


def _native(x):
    n = x.shape[0]
    return x.transpose(0, 1, 3, 2).reshape(n, SUBS, LANES)


def kernel(anchor_query_prob, aux_query_prob, anchor_voxel_prob,
           aux_voxel_prob, ensemble_sem_prob_dense):
    del anchor_query_prob, aux_query_prob
    a3 = _native(anchor_voxel_prob)
    b3 = _native(aux_voxel_prob)
    s3 = _native(ensemble_sem_prob_dense)

    ta = (a3.reshape(QN, KTOT).T > 0.5).astype(jnp.int8)
    tb = (b3.reshape(QN, KTOT).T > 0.5).astype(jnp.int8)

    mask, idx, scale = _pass_a(ta, tb, s3)
    out = _pass_b(idx.reshape(QPAD), scale.reshape(QPAD), a3, b3, mask)
    return out.reshape(QN, 128, 16, 128).transpose(0, 1, 3, 2)
